# baseline (speedup 1.0000x reference)
import jax
import jax.numpy as jnp
from jax.experimental import pallas as pl
from jax.experimental.pallas import tpu as pltpu

N, D, E, DFF = 2048, 1024, 8, 1024
NL = N // 2

_INTERPRET = False


def _ln(t, g, b, eps=1e-5):
    m = jnp.mean(t, axis=-1, keepdims=True)
    v = jnp.mean((t - m) ** 2, axis=-1, keepdims=True)
    return (t - m) * jax.lax.rsqrt(v + eps) * g + b


def _top2_gates(logits):
    n, e = logits.shape
    iota = jax.lax.broadcasted_iota(jnp.int32, (n, e), 1)
    m0 = jnp.max(logits, axis=-1, keepdims=True)
    i0 = jnp.min(jnp.where(logits == m0, iota, e), axis=-1, keepdims=True)
    masked = jnp.where(iota == i0, -jnp.inf, logits)
    m1 = jnp.max(masked, axis=-1, keepdims=True)
    i1 = jnp.min(jnp.where(masked == m1, iota, e), axis=-1, keepdims=True)
    g1 = jnp.exp(m1 - m0)
    g0 = 1.0 / (1.0 + g1)
    g1 = 1.0 - g0
    return jnp.where(iota == i0, g0, 0.0) + jnp.where(iota == i1, g1, 0.0)


def _down_body(x_ref, pw_ref, W_ref, b_ref, g_ref, be_ref, lng_ref, lnb_ref,
               Wr_ref, br_ref, xd_ref, h_ref, gates_ref):
    x = x_ref[...]
    w = jax.nn.softmax(pw_ref[...], axis=-1)
    xg = x.reshape(NL, 2, D)
    xp = xg[:, 0, :] * w[:, 0:1] + xg[:, 1, :] * w[:, 1:2]
    t = jnp.dot(xp, W_ref[...], preferred_element_type=jnp.float32) + b_ref[...]
    xd = jnp.maximum(_ln(t, g_ref[...], be_ref[...]), 0.0)
    xd_ref[...] = xd
    h = _ln(xd, lng_ref[...], lnb_ref[...])
    h_ref[...] = h
    logits = jnp.dot(h, Wr_ref[...], preferred_element_type=jnp.float32) + br_ref[...]
    gates_ref[...] = _top2_gates(logits)


def _moe_lo_body(xd_ref, h_ref, gates_ref, W1_ref, b1_ref, W2_ref, b2_ref, out_ref):
    e = pl.program_id(0)
    h = h_ref[...]
    hidden = jnp.maximum(
        jnp.dot(h, W1_ref[0], preferred_element_type=jnp.float32) + b1_ref[0], 0.0)
    y = jnp.dot(hidden, W2_ref[0], preferred_element_type=jnp.float32) + b2_ref[0]
    eo = (jax.lax.broadcasted_iota(jnp.int32, (1, E), 1) == e).astype(jnp.float32)
    g = jnp.sum(gates_ref[...] * eo, axis=-1, keepdims=True)
    contrib = g * y

    @pl.when(e == 0)
    def _():
        out_ref[...] = xd_ref[...] + contrib

    @pl.when(e != 0)
    def _():
        out_ref[...] = out_ref[...] + contrib


def _up_body(xl_ref, W1_ref, b1_ref, g_ref, be_ref, W2_ref, b2_ref, pos_ref,
             xe_ref, xo_ref, lng_ref, lnb_ref, Wr_ref, br_ref, s_ref,
             xce_ref, xco_ref, he_ref, ho_ref, ge_ref, go_ref):
    xl = xl_ref[...]
    t1 = jnp.dot(xl, W1_ref[...], preferred_element_type=jnp.float32) + b1_ref[...]
    t1 = jnp.maximum(_ln(t1, g_ref[...], be_ref[...]), 0.0)
    t2 = jnp.dot(t1, W2_ref[...], preferred_element_type=jnp.float32) + b2_ref[...]
    s = s_ref[0, 0]
    xce = xe_ref[...] + s * (t2[:, :D] + pos_ref[0:1, :])
    xco = xo_ref[...] + s * (t2[:, D:] + pos_ref[1:2, :])
    xce_ref[...] = xce
    xco_ref[...] = xco
    he = _ln(xce, lng_ref[...], lnb_ref[...])
    ho = _ln(xco, lng_ref[...], lnb_ref[...])
    he_ref[...] = he
    ho_ref[...] = ho
    ge_ref[...] = _top2_gates(
        jnp.dot(he, Wr_ref[...], preferred_element_type=jnp.float32) + br_ref[...])
    go_ref[...] = _top2_gates(
        jnp.dot(ho, Wr_ref[...], preferred_element_type=jnp.float32) + br_ref[...])


def _moe_hi_body(xpm_ref, xcp_ref, hp_ref, gp_ref, W1_ref, b1_ref, W2_ref, b2_ref,
                 s_ref, out_ref):
    e = pl.program_id(1)
    h = hp_ref[...]
    hidden = jnp.maximum(
        jnp.dot(h, W1_ref[0], preferred_element_type=jnp.float32) + b1_ref[0], 0.0)
    y = jnp.dot(hidden, W2_ref[0], preferred_element_type=jnp.float32) + b2_ref[0]
    eo = (jax.lax.broadcasted_iota(jnp.int32, (1, E), 1) == e).astype(jnp.float32)
    g = jnp.sum(gp_ref[...] * eo, axis=-1, keepdims=True)
    s = s_ref[0, 0]
    contrib = s * g * y

    @pl.when(e == 0)
    def _():
        out_ref[...] = xpm_ref[...] + s * xcp_ref[...] + contrib

    @pl.when(e != 0)
    def _():
        out_ref[...] = out_ref[...] + contrib


def _full(shape):
    nd = len(shape)
    return pl.BlockSpec(shape, lambda *_: (0,) * nd)


def kernel(x, down_pool_w, down_W, down_b, down_g, down_beta, lo_ln_g, lo_ln_b,
           lo_Wr, lo_br, lo_W1, lo_b1, lo_W2, lo_b2, up_W1, up_b1, up_g, up_beta,
           up_W2, up_b2, up_pos, hi_ln_g, hi_ln_b, hi_Wr, hi_br, hi_W1, hi_b1,
           hi_W2, hi_b2, scale_lo, scale_hi):
    f32 = jnp.float32
    x2 = x[0]
    r2 = lambda v: v.reshape(1, -1)
    sL = jnp.reshape(scale_lo, (1, 1)).astype(f32)
    sH = jnp.reshape(scale_hi, (1, 1)).astype(f32)

    xd, h_lo, gates_lo = pl.pallas_call(
        _down_body,
        out_shape=[jax.ShapeDtypeStruct((NL, D), f32),
                   jax.ShapeDtypeStruct((NL, D), f32),
                   jax.ShapeDtypeStruct((NL, E), f32)],
        interpret=_INTERPRET,
    )(x2, down_pool_w, down_W, r2(down_b), r2(down_g), r2(down_beta),
      r2(lo_ln_g), r2(lo_ln_b), lo_Wr, r2(lo_br))

    xl = pl.pallas_call(
        _moe_lo_body,
        grid=(E,),
        in_specs=[
            pl.BlockSpec((NL, D), lambda e: (0, 0)),
            pl.BlockSpec((NL, D), lambda e: (0, 0)),
            pl.BlockSpec((NL, E), lambda e: (0, 0)),
            pl.BlockSpec((1, D, DFF), lambda e: (e, 0, 0)),
            pl.BlockSpec((1, 1, DFF), lambda e: (e, 0, 0)),
            pl.BlockSpec((1, DFF, D), lambda e: (e, 0, 0)),
            pl.BlockSpec((1, 1, D), lambda e: (e, 0, 0)),
        ],
        out_specs=pl.BlockSpec((NL, D), lambda e: (0, 0)),
        out_shape=jax.ShapeDtypeStruct((NL, D), f32),
        interpret=_INTERPRET,
    )(xd, h_lo, gates_lo, lo_W1, lo_b1.reshape(E, 1, DFF), lo_W2,
      lo_b2.reshape(E, 1, D))

    x3 = x2.reshape(NL, 2, D)
    xe, xo = x3[:, 0, :], x3[:, 1, :]

    BU = 256
    TU = NL // BU
    up_outs = pl.pallas_call(
        _up_body,
        grid=(TU,),
        in_specs=[
            pl.BlockSpec((BU, D), lambda t: (t, 0)),
            _full((D, 2 * D)),
            _full((1, 2 * D)),
            _full((1, 2 * D)),
            _full((1, 2 * D)),
            _full((2 * D, 2 * D)),
            _full((1, 2 * D)),
            _full((2, D)),
            pl.BlockSpec((BU, D), lambda t: (t, 0)),
            pl.BlockSpec((BU, D), lambda t: (t, 0)),
            _full((1, D)),
            _full((1, D)),
            _full((D, E)),
            _full((1, E)),
            _full((1, 1)),
        ],
        out_specs=[
            pl.BlockSpec((BU, D), lambda t: (t, 0)),
            pl.BlockSpec((BU, D), lambda t: (t, 0)),
            pl.BlockSpec((BU, D), lambda t: (t, 0)),
            pl.BlockSpec((BU, D), lambda t: (t, 0)),
            pl.BlockSpec((BU, E), lambda t: (t, 0)),
            pl.BlockSpec((BU, E), lambda t: (t, 0)),
        ],
        out_shape=[jax.ShapeDtypeStruct((NL, D), f32),
                   jax.ShapeDtypeStruct((NL, D), f32),
                   jax.ShapeDtypeStruct((NL, D), f32),
                   jax.ShapeDtypeStruct((NL, D), f32),
                   jax.ShapeDtypeStruct((NL, E), f32),
                   jax.ShapeDtypeStruct((NL, E), f32)],
        interpret=_INTERPRET,
    )(xl, up_W1, r2(up_b1), r2(up_g), r2(up_beta), up_W2, r2(up_b2), up_pos,
      xe, xo, r2(hi_ln_g), r2(hi_ln_b), hi_Wr, r2(hi_br), sL)
    xce, xco, he, ho, ge, go = up_outs

    xpm = jnp.concatenate([xe, xo], axis=0)
    xcp = jnp.concatenate([xce, xco], axis=0)
    hp = jnp.concatenate([he, ho], axis=0)
    gp = jnp.concatenate([ge, go], axis=0)

    BM = 512
    TM = N // BM
    outp = pl.pallas_call(
        _moe_hi_body,
        grid=(TM, E),
        in_specs=[
            pl.BlockSpec((BM, D), lambda t, e: (t, 0)),
            pl.BlockSpec((BM, D), lambda t, e: (t, 0)),
            pl.BlockSpec((BM, D), lambda t, e: (t, 0)),
            pl.BlockSpec((BM, E), lambda t, e: (t, 0)),
            pl.BlockSpec((1, D, DFF), lambda t, e: (e, 0, 0)),
            pl.BlockSpec((1, 1, DFF), lambda t, e: (e, 0, 0)),
            pl.BlockSpec((1, DFF, D), lambda t, e: (e, 0, 0)),
            pl.BlockSpec((1, 1, D), lambda t, e: (e, 0, 0)),
            pl.BlockSpec((1, 1), lambda t, e: (0, 0)),
        ],
        out_specs=pl.BlockSpec((BM, D), lambda t, e: (t, 0)),
        out_shape=jax.ShapeDtypeStruct((N, D), f32),
        interpret=_INTERPRET,
    )(xpm, xcp, hp, gp, hi_W1, hi_b1.reshape(E, 1, DFF), hi_W2,
      hi_b2.reshape(E, 1, D), sH)

    return outp.reshape(2, NL, D).transpose(1, 0, 2).reshape(1, N, D)
